# Initial kernel scaffold; baseline (speedup 1.0000x reference)
import functools

import jax
import jax.numpy as jnp
from jax import lax
from jax.experimental import pallas as pl
from jax.experimental.pallas import tpu as pltpu
from jax.experimental.pallas import tpu_sc as plsc

N = 10000
E = 320000
D = 128
LAYERS = 3
NC = 2
NS = 16
NW = NC * NS
CH = 128
ROWS = E // CH
RPN = N // NS

_MESH = plsc.VectorSubcoreMesh(core_axis_name="c", subcore_axis_name="s")


@functools.partial(
    pl.kernel,
    out_type=jax.ShapeDtypeStruct((NC, N, D), jnp.float32),
    mesh=_MESH,
    scratch_types=[
        pltpu.VMEM_SHARED((N, D), jnp.float32),
        pltpu.VMEM((CH,), jnp.int32),
        pltpu.VMEM((CH,), jnp.int32),
        pltpu.VMEM((CH, D), jnp.float32),
        pltpu.SemaphoreType.DMA,
    ],
)
def _sc_agg(vals_hbm, src_hbm, dst_hbm, zero_hbm, out_hbm,
            acc, sidx, didx, rows, sem):
    c = lax.axis_index("c")
    s = lax.axis_index("s")
    k = c * NS + s
    pltpu.sync_copy(zero_hbm.at[pl.ds(s * RPN, RPN)], acc.at[pl.ds(s * RPN, RPN)])
    plsc.subcore_barrier()
    lo = (k * ROWS) // NW
    hi = ((k + 1) * ROWS) // NW

    def step(j, carry):
        pltpu.sync_copy(src_hbm.at[j], sidx)
        pltpu.sync_copy(dst_hbm.at[j], didx)
        pltpu.async_copy(vals_hbm.at[sidx], rows, sem).wait()
        pltpu.sync_copy(rows, acc.at[didx], add=True)
        return carry

    lax.fori_loop(lo, hi, step, 0)
    plsc.subcore_barrier()
    pltpu.sync_copy(acc.at[pl.ds(s * RPN, RPN)],
                    out_hbm.at[c, pl.ds(s * RPN, RPN)])


@functools.partial(
    pl.kernel,
    out_type=jax.ShapeDtypeStruct((NC, N, 16), jnp.float32),
    mesh=_MESH,
    scratch_types=[
        pltpu.VMEM_SHARED((N, 16), jnp.float32),
        pltpu.VMEM((CH,), jnp.int32),
        pltpu.VMEM((CH, 16), jnp.float32),
    ],
)
def _sc_deg(dst_hbm, ones_hbm, zero_hbm, out_hbm, acc, didx, ones):
    c = lax.axis_index("c")
    s = lax.axis_index("s")
    k = c * NS + s
    pltpu.sync_copy(ones_hbm, ones)
    pltpu.sync_copy(zero_hbm.at[pl.ds(s * RPN, RPN)], acc.at[pl.ds(s * RPN, RPN)])
    plsc.subcore_barrier()
    lo = (k * ROWS) // NW
    hi = ((k + 1) * ROWS) // NW

    def step(j, carry):
        pltpu.sync_copy(dst_hbm.at[j], didx)
        pltpu.sync_copy(ones, acc.at[didx], add=True)
        return carry

    lax.fori_loop(lo, hi, step, 0)
    plsc.subcore_barrier()
    pltpu.sync_copy(acc.at[pl.ds(s * RPN, RPN)],
                    out_hbm.at[c, pl.ds(s * RPN, RPN)])


def _dot(a, b):
    return lax.dot_general(a, b, (((1,), (0,)), ((), ())),
                           precision=lax.Precision.HIGHEST,
                           preferred_element_type=jnp.float32)


def _enc_body(x_ref, w_ref, b_ref, o_ref):
    o_ref[...] = jnp.maximum(_dot(x_ref[...], w_ref[...]) + b_ref[...], 0.0)


_enc = pl.pallas_call(
    _enc_body, out_shape=jax.ShapeDtypeStruct((N, D), jnp.float32))


def _layer_body(residual, h_ref, p_ref, degp_ref, w1h_ref, w1a_ref, b1_ref,
                w2_ref, b2_ref, o_ref):
    deg = degp_ref[:, 0:1] + degp_ref[:, 1:2]
    inv = 1.0 / jnp.maximum(deg, 1.0)
    agg = (p_ref[0] + p_ref[1]) * inv
    z = jnp.maximum(_dot(h_ref[...], w1h_ref[...]) +
                    _dot(agg, w1a_ref[...]) + b1_ref[...], 0.0)
    hn = jnp.maximum(_dot(z, w2_ref[...]) + b2_ref[...], 0.0)
    if residual:
        hn = hn + h_ref[...]
    o_ref[...] = hn


_layer_first = pl.pallas_call(
    functools.partial(_layer_body, False),
    out_shape=jax.ShapeDtypeStruct((N, D), jnp.float32))
_layer_res = pl.pallas_call(
    functools.partial(_layer_body, True),
    out_shape=jax.ShapeDtypeStruct((N, D), jnp.float32))


def _head_body(h_ref, w_ref, b_ref, o_ref):
    pooled = jnp.mean(h_ref[...], axis=0, keepdims=True)
    o_ref[...] = _dot(pooled, w_ref[...]) + b_ref[...]


_head = pl.pallas_call(
    _head_body, out_shape=jax.ShapeDtypeStruct((1, D), jnp.float32))


def kernel(x, edge_index, enc_W, enc_b, conv_W1, conv_b1, conv_W2, conv_b2,
           head_W, head_b):
    ei = edge_index.astype(jnp.int32)
    src2 = ei[0].reshape(ROWS, CH)
    dst2 = ei[1].reshape(ROWS, CH)
    zero_d = jnp.zeros((N, D), jnp.float32)
    zero_16 = jnp.zeros((N, 16), jnp.float32)
    ones_16 = jnp.ones((CH, 16), jnp.float32)

    h = _enc(x, enc_W, enc_b.reshape(1, D))
    degp = _sc_deg(dst2, ones_16, zero_16)
    degp = jnp.moveaxis(degp[:, :, 0], 0, 1)
    for i in range(LAYERS):
        part = _sc_agg(h, src2, dst2, zero_d)
        layer = _layer_first if i == 0 else _layer_res
        h = layer(h, part, degp,
                  conv_W1[i, :D], conv_W1[i, D:], conv_b1[i].reshape(1, -1),
                  conv_W2[i], conv_b2[i].reshape(1, -1))
    out = _head(h, head_W, head_b.reshape(1, D))
    return out.reshape(D)


# speedup vs baseline: 5.5988x; 5.5988x over previous
import functools

import jax
import jax.numpy as jnp
from jax import lax
from jax.experimental import pallas as pl
from jax.experimental.pallas import tpu as pltpu
from jax.experimental.pallas import tpu_sc as plsc

N = 10000
E = 320000
D = 128
LAYERS = 3
NC = 2
NS = 16
NW = NC * NS
CH = 128
CHUNKS = E // CH
NP = 10240
SRP = NP // NS


@functools.lru_cache(maxsize=None)
def _sc_kernels():
    mesh = plsc.VectorSubcoreMesh(core_axis_name="c", subcore_axis_name="s",
                                  num_cores=NC, num_subcores=NS)

    @functools.partial(
        pl.kernel,
        out_type=jax.ShapeDtypeStruct((NC, NP, D), jnp.float32),
        mesh=mesh,
        scratch_types=[
            pltpu.VMEM_SHARED((NP, D), jnp.float32),
            pltpu.VMEM((CH,), jnp.int32),
            pltpu.VMEM((CH,), jnp.int32),
            pltpu.VMEM((CH, D), jnp.float32),
            pltpu.SemaphoreType.DMA,
        ],
    )
    def sc_agg(vals_hbm, src_hbm, dst_hbm, zero_hbm, out_hbm,
               acc, sidx, didx, rows, sem):
        c = lax.axis_index("c")
        s = lax.axis_index("s")
        k = c * NS + s
        pltpu.sync_copy(zero_hbm.at[pl.ds(s * SRP, SRP)],
                        acc.at[pl.ds(s * SRP, SRP)])
        plsc.subcore_barrier()
        lo = (k * CHUNKS) // NW
        hi = ((k + 1) * CHUNKS) // NW

        def step(j, carry):
            pltpu.sync_copy(src_hbm.at[pl.ds(j * CH, CH)], sidx)
            pltpu.sync_copy(dst_hbm.at[pl.ds(j * CH, CH)], didx)
            pltpu.async_copy(vals_hbm.at[sidx], rows, sem).wait()
            pltpu.sync_copy(rows, acc.at[didx], add=True)
            return carry

        lax.fori_loop(lo, hi, step, 0)
        plsc.subcore_barrier()
        pltpu.sync_copy(acc.at[pl.ds(s * SRP, SRP)],
                        out_hbm.at[c, pl.ds(s * SRP, SRP)])

    @functools.partial(
        pl.kernel,
        out_type=jax.ShapeDtypeStruct((NC, NP, D), jnp.float32),
        mesh=mesh,
        scratch_types=[
            pltpu.VMEM_SHARED((NP, D), jnp.float32),
            pltpu.VMEM((CH,), jnp.int32),
            pltpu.VMEM((CH, D), jnp.float32),
        ],
    )
    def sc_deg(dst_hbm, zero_hbm, out_hbm, acc, didx, ones):
        c = lax.axis_index("c")
        s = lax.axis_index("s")
        k = c * NS + s

        def fill(j, carry):
            ones[j, pl.ds(0, 16)] = jnp.ones((16,), jnp.float32)
            return carry

        lax.fori_loop(0, CH, fill, 0)
        pltpu.sync_copy(zero_hbm.at[pl.ds(s * SRP, SRP)],
                        acc.at[pl.ds(s * SRP, SRP)])
        plsc.subcore_barrier()
        lo = (k * CHUNKS) // NW
        hi = ((k + 1) * CHUNKS) // NW

        def step(j, carry):
            pltpu.sync_copy(dst_hbm.at[pl.ds(j * CH, CH)], didx)
            pltpu.sync_copy(ones, acc.at[didx], add=True)
            return carry

        lax.fori_loop(lo, hi, step, 0)
        plsc.subcore_barrier()
        pltpu.sync_copy(acc.at[pl.ds(s * SRP, SRP)],
                        out_hbm.at[c, pl.ds(s * SRP, SRP)])

    return sc_agg, sc_deg


def _dot(a, b):
    return lax.dot_general(a, b, (((1,), (0,)), ((), ())),
                           precision=lax.Precision.HIGHEST,
                           preferred_element_type=jnp.float32)


def _enc_body(x_ref, w_ref, b_ref, o_ref):
    o_ref[...] = jnp.maximum(_dot(x_ref[...], w_ref[...]) + b_ref[...], 0.0)


_enc = pl.pallas_call(
    _enc_body, out_shape=jax.ShapeDtypeStruct((N, D), jnp.float32))


def _layer_body(residual, h_ref, p_ref, degp_ref, w1h_ref, w1a_ref, b1_ref,
                w2_ref, b2_ref, o_ref):
    deg = degp_ref[0, :N, 0:1] + degp_ref[1, :N, 0:1]
    inv = 1.0 / jnp.maximum(deg, 1.0)
    agg = (p_ref[0, :N] + p_ref[1, :N]) * inv
    z = jnp.maximum(_dot(h_ref[...], w1h_ref[...]) +
                    _dot(agg, w1a_ref[...]) + b1_ref[...], 0.0)
    hn = jnp.maximum(_dot(z, w2_ref[...]) + b2_ref[...], 0.0)
    if residual:
        hn = hn + h_ref[...]
    o_ref[...] = hn


_layer_first = pl.pallas_call(
    functools.partial(_layer_body, False),
    out_shape=jax.ShapeDtypeStruct((N, D), jnp.float32))
_layer_res = pl.pallas_call(
    functools.partial(_layer_body, True),
    out_shape=jax.ShapeDtypeStruct((N, D), jnp.float32))


def _head_body(h_ref, w_ref, b_ref, o_ref):
    pooled = jnp.mean(h_ref[...], axis=0, keepdims=True)
    o_ref[...] = _dot(pooled, w_ref[...]) + b_ref[...]


_head = pl.pallas_call(
    _head_body, out_shape=jax.ShapeDtypeStruct((1, D), jnp.float32))


def kernel(x, edge_index, enc_W, enc_b, conv_W1, conv_b1, conv_W2, conv_b2,
           head_W, head_b):
    ei = edge_index.astype(jnp.int32)
    src1 = ei[0]
    dst1 = ei[1]
    zero_d = jnp.zeros((NP, D), jnp.float32)

    sc_agg, sc_deg = _sc_kernels()
    h = _enc(x, enc_W, enc_b.reshape(1, D))
    degp = sc_deg(dst1, zero_d)
    for i in range(LAYERS):
        part = sc_agg(h, src1, dst1, zero_d)
        layer = _layer_first if i == 0 else _layer_res
        h = layer(h, part, degp,
                  conv_W1[i, :D], conv_W1[i, D:], conv_b1[i].reshape(1, -1),
                  conv_W2[i], conv_b2[i].reshape(1, -1))
    out = _head(h, head_W, head_b.reshape(1, D))
    return out.reshape(D)
